# Initial kernel scaffold; baseline (speedup 1.0000x reference)
import jax
import jax.numpy as jnp
from jax.experimental import pallas as pl


def kernel(x, edge_index, ts_target, node_target, node_mask, W1, b1, W2, b2, Wt1, bt1, Wt2, bt2, Wc1, bc1, Wc2, bc2):
    raise NotImplementedError("write your pallas kernel here")


# speedup vs baseline: 8.9815x; 8.9815x over previous
import functools

import jax
import jax.numpy as jnp
from jax import lax
from jax.experimental import pallas as pl
from jax.experimental.pallas import tpu as pltpu
from jax.experimental.pallas import tpu_sc as plsc

N = 10000
E = 320000
D = 128
NTS = 5

NC = 2
NS = 16
NW = NC * NS
CH = 128
NCHUNK = 80
EPW = CH * NCHUNK
EPAD = EPW * NW
DUMP = N
NACC = 10240
RPT = NACC // NS
MCH = 2
MPAD = MCH * CH * NW

WIN = 16

RB = 2000


@functools.cache
def _sc_kernels():
    mesh = plsc.VectorSubcoreMesh(
        core_axis_name="c", subcore_axis_name="s", num_cores=NC, num_subcores=NS)

    @functools.partial(
        pl.kernel,
        out_type=jax.ShapeDtypeStruct((NC, NACC, D), jnp.float32),
        mesh=mesh,
        scratch_types=[
            pltpu.VMEM((NCHUNK, CH), jnp.int32),
            pltpu.VMEM((MCH, CH), jnp.int32),
            pltpu.VMEM((CH, D), jnp.float32),
            pltpu.VMEM((CH, D), jnp.float32),
            pltpu.VMEM_SHARED((NACC, D), jnp.float32),
        ],
    )
    def hist(dst_hbm, midx_hbm, ones_hbm, vm_hbm, zrow_hbm,
             tab_out,
             dst_v, midx_v, val_v, zb_v, tab_sp):
        c = lax.axis_index("c")
        s = lax.axis_index("s")
        w = c * NS + s
        pltpu.sync_copy(dst_hbm.at[w], dst_v)
        pltpu.sync_copy(midx_hbm.at[w], midx_v)
        pltpu.sync_copy(zrow_hbm, zb_v)
        base = s * RPT

        def zero_blk(k, carry):
            pltpu.sync_copy(zb_v, tab_sp.at[pl.ds(base + k * CH, CH)])
            return carry

        lax.fori_loop(0, RPT // CH, zero_blk, 0)
        pltpu.sync_copy(ones_hbm, val_v)
        plsc.subcore_barrier()

        def deg_blk(j, carry):
            pltpu.sync_copy(val_v, tab_sp.at[dst_v.at[j]], add=True)
            return carry

        lax.fori_loop(0, NCHUNK, deg_blk, 0)

        def mask_blk(j, carry):
            pltpu.sync_copy(vm_hbm.at[w, j], val_v)
            pltpu.sync_copy(val_v, tab_sp.at[midx_v.at[j]], add=True)
            return carry

        lax.fori_loop(0, MCH, mask_blk, 0)
        plsc.subcore_barrier()
        pltpu.sync_copy(tab_sp.at[pl.ds(base, RPT)],
                        tab_out.at[c, pl.ds(base, RPT)])

    @functools.partial(
        pl.kernel,
        out_type=jax.ShapeDtypeStruct((NC, NACC, D), jnp.float32),
        mesh=mesh,
        scratch_types=[
            pltpu.VMEM((WIN, CH), jnp.int32),
            pltpu.VMEM((WIN, CH), jnp.int32),
            pltpu.VMEM((CH, D), jnp.float32),
            pltpu.VMEM((CH, D), jnp.float32),
            pltpu.VMEM_SHARED((NACC, D), jnp.float32),
            pltpu.SemaphoreType.DMA,
            pltpu.SemaphoreType.DMA,
        ],
    )
    def segsum(tbl_hbm, src_hbm, dst_hbm, zrow_hbm, out_hbm,
               srcw_v, dstw_v, rows0, rows1, acc_sp, sem0, sem1):
        c = lax.axis_index("c")
        s = lax.axis_index("s")
        w = c * NS + s
        pltpu.sync_copy(zrow_hbm, rows0)
        base = s * RPT

        def zero_blk(k, carry):
            pltpu.sync_copy(rows0, acc_sp.at[pl.ds(base + k * CH, CH)])
            return carry

        lax.fori_loop(0, RPT // CH, zero_blk, 0)
        plsc.subcore_barrier()

        def window(win, carry):
            pltpu.sync_copy(src_hbm.at[w, pl.ds(win * WIN, WIN)], srcw_v)
            pltpu.sync_copy(dst_hbm.at[w, pl.ds(win * WIN, WIN)], dstw_v)
            pltpu.async_copy(tbl_hbm.at[srcw_v.at[0]], rows0, sem0)
            pltpu.async_copy(tbl_hbm.at[srcw_v.at[1]], rows1, sem1)

            def pair(jj, c2):
                a = 2 * jj
                b = a + 1
                pltpu.make_async_copy(
                    tbl_hbm.at[srcw_v.at[a]], rows0, sem0).wait()
                pltpu.sync_copy(rows0, acc_sp.at[dstw_v.at[a]], add=True)

                @pl.when(a + 2 < WIN)
                def _():
                    pltpu.async_copy(tbl_hbm.at[srcw_v.at[a + 2]], rows0, sem0)

                pltpu.make_async_copy(
                    tbl_hbm.at[srcw_v.at[b]], rows1, sem1).wait()
                pltpu.sync_copy(rows1, acc_sp.at[dstw_v.at[b]], add=True)

                @pl.when(b + 2 < WIN)
                def _():
                    pltpu.async_copy(tbl_hbm.at[srcw_v.at[b + 2]], rows1, sem1)

                return c2

            lax.fori_loop(0, WIN // 2, pair, 0)
            return carry

        lax.fori_loop(0, NCHUNK // WIN, window, 0)
        plsc.subcore_barrier()
        pltpu.sync_copy(acc_sp.at[pl.ds(base, RPT)],
                        out_hbm.at[c, pl.ds(base, RPT)])

    return hist, segsum


def _dinv_of(deg_ref):
    deg = deg_ref[0, :, 0:1] + deg_ref[1, :, 0:1] + 1.0
    return lax.rsqrt(deg)


def _h1_body(x_ref, w1_ref, deg_ref, out_ref):
    dinv = _dinv_of(deg_ref)
    h = jnp.dot(x_ref[:], w1_ref[:], preferred_element_type=jnp.float32)
    out_ref[:] = h * dinv


def _mid_body(acc_ref, h1_ref, deg_ref, b1_ref, w2_ref, out_ref):
    dinv = _dinv_of(deg_ref)
    zsum = acc_ref[0] + acc_ref[1] + h1_ref[:]
    z = jnp.maximum(dinv * zsum + b1_ref[:], 0.0)
    h2 = jnp.dot(z, w2_ref[:], preferred_element_type=jnp.float32)
    out_ref[:] = h2 * dinv


def _loss_body(acc_ref, h2_ref, deg_ref, tst_ref,
               b2_ref, wt1_ref, bt1_ref, wt2_ref, bt2_ref,
               wc1_ref, bc1_ref, wc2_ref, bc2_ref,
               cls_ref, ts_ref):
    i = pl.program_id(0)
    dinv = _dinv_of(deg_ref)
    zsum = acc_ref[0] + acc_ref[1] + h2_ref[:]
    z2 = dinv * zsum + b2_ref[:]

    col = lax.broadcasted_iota(jnp.int32, (1, D), 1)
    neg = jnp.float32(-1e30)

    t1 = jnp.maximum(
        jnp.dot(z2, wt1_ref[:], preferred_element_type=jnp.float32)
        + bt1_ref[:], 0.0)
    tl = jnp.dot(t1, wt2_ref[:], preferred_element_type=jnp.float32) + bt2_ref[:]
    tlm = jnp.where(col < NTS, tl, neg)
    m = jnp.max(tlm, axis=1, keepdims=True)
    se = jnp.sum(jnp.exp(tlm - m), axis=1, keepdims=True)
    lse = m + jnp.log(se)
    picked = jnp.sum(jnp.where(col == tst_ref[:], tl, 0.0), axis=1,
                     keepdims=True)
    ts_part = jnp.sum(lse - picked) * (1.0 / N)

    c1h = jnp.maximum(
        jnp.dot(z2, wc1_ref[:], preferred_element_type=jnp.float32)
        + bc1_ref[:], 0.0)
    cl = jnp.dot(c1h, wc2_ref[:], preferred_element_type=jnp.float32) + bc2_ref[:]
    clm = jnp.where(col < 2, cl, neg)
    m2 = jnp.max(clm, axis=1, keepdims=True)
    se2 = jnp.sum(jnp.exp(clm - m2), axis=1, keepdims=True)
    lse2 = m2 + jnp.log(se2)
    c0 = deg_ref[0, :, 1:2] + deg_ref[1, :, 1:2]
    c1 = deg_ref[0, :, 2:3] + deg_ref[1, :, 2:3]
    cls_part = jnp.sum((c0 + c1) * lse2 - c0 * cl[:, 0:1]
                       - c1 * cl[:, 1:2]) * (1.0 / 5000.0)

    @pl.when(i == 0)
    def _():
        cls_ref[:, :] = jnp.zeros((1, 1), jnp.float32)
        ts_ref[:, :] = jnp.zeros((1, 1), jnp.float32)

    cls_ref[:, :] += cls_part.reshape(1, 1)
    ts_ref[:, :] += ts_part.reshape(1, 1)


_h1_call = pl.pallas_call(
    _h1_body,
    grid=(N // RB,),
    in_specs=[
        pl.BlockSpec((RB, D), lambda i: (i, 0)),
        pl.BlockSpec((D, D), lambda i: (0, 0)),
        pl.BlockSpec((NC, RB, D), lambda i: (0, i, 0)),
    ],
    out_specs=pl.BlockSpec((RB, D), lambda i: (i, 0)),
    out_shape=jax.ShapeDtypeStruct((N, D), jnp.float32),
)

_mid_call = pl.pallas_call(
    _mid_body,
    grid=(N // RB,),
    in_specs=[
        pl.BlockSpec((NC, RB, D), lambda i: (0, i, 0)),
        pl.BlockSpec((RB, D), lambda i: (i, 0)),
        pl.BlockSpec((NC, RB, D), lambda i: (0, i, 0)),
        pl.BlockSpec((1, D), lambda i: (0, 0)),
        pl.BlockSpec((D, D), lambda i: (0, 0)),
    ],
    out_specs=pl.BlockSpec((RB, D), lambda i: (i, 0)),
    out_shape=jax.ShapeDtypeStruct((N, D), jnp.float32),
)

_loss_call = pl.pallas_call(
    _loss_body,
    grid=(N // RB,),
    in_specs=[
        pl.BlockSpec((NC, RB, D), lambda i: (0, i, 0)),
        pl.BlockSpec((RB, D), lambda i: (i, 0)),
        pl.BlockSpec((NC, RB, D), lambda i: (0, i, 0)),
        pl.BlockSpec((RB, 1), lambda i: (i, 0)),
        pl.BlockSpec((1, D), lambda i: (0, 0)),
        pl.BlockSpec((D, D), lambda i: (0, 0)),
        pl.BlockSpec((1, D), lambda i: (0, 0)),
        pl.BlockSpec((D, D), lambda i: (0, 0)),
        pl.BlockSpec((1, D), lambda i: (0, 0)),
        pl.BlockSpec((D, D), lambda i: (0, 0)),
        pl.BlockSpec((1, D), lambda i: (0, 0)),
        pl.BlockSpec((D, D), lambda i: (0, 0)),
        pl.BlockSpec((1, D), lambda i: (0, 0)),
    ],
    out_specs=[
        pl.BlockSpec((1, 1), lambda i: (0, 0)),
        pl.BlockSpec((1, 1), lambda i: (0, 0)),
    ],
    out_shape=[
        jax.ShapeDtypeStruct((1, 1), jnp.float32),
        jax.ShapeDtypeStruct((1, 1), jnp.float32),
    ],
)


def kernel(x, edge_index, ts_target, node_target, node_mask,
           W1, b1, W2, b2, Wt1, bt1, Wt2, bt2, Wc1, bc1, Wc2, bc2):
    f32 = jnp.float32
    i32 = jnp.int32
    hist, segsum = _sc_kernels()

    src = edge_index[0]
    dst = edge_index[1]
    epad = EPAD - E
    srcp = jnp.concatenate([src, jnp.zeros((epad,), i32)]).reshape(
        NW, NCHUNK, CH)
    dstp = jnp.concatenate([dst, jnp.full((epad,), DUMP, i32)]).reshape(
        NW, NCHUNK, CH)

    mpad = MPAD - node_mask.shape[0]
    midx = jnp.concatenate([node_mask, jnp.full((mpad,), DUMP, i32)]).reshape(
        NW, MCH, CH)
    lanes = jnp.arange(D, dtype=i32)
    lane0 = (lanes == 0).astype(f32)
    lane1 = (lanes == 1).astype(f32)
    lane2 = (lanes == 2).astype(f32)
    v0 = jnp.concatenate([(node_target == 0).astype(f32),
                          jnp.zeros((mpad,), f32)])
    v1 = jnp.concatenate([(node_target == 1).astype(f32),
                          jnp.zeros((mpad,), f32)])
    vmr = (v0[:, None] * lane1 + v1[:, None] * lane2).reshape(NW, MCH, CH, D)
    ones_rows = jnp.broadcast_to(lane0, (CH, D))
    zrow = jnp.zeros((CH, D), f32)

    deg_t = hist(dstp, midx, ones_rows, vmr, zrow)

    h1 = _h1_call(x, W1, deg_t)
    acc1 = segsum(h1, srcp, dstp, zrow)
    h2 = _mid_call(acc1, h1, deg_t, b1.reshape(1, D), W2)
    acc2 = segsum(h2, srcp, dstp, zrow)

    wt2p = jnp.zeros((D, D), f32).at[:, :NTS].set(Wt2)
    bt2p = jnp.zeros((1, D), f32).at[0, :NTS].set(bt2)
    wc2p = jnp.zeros((D, D), f32).at[:, :2].set(Wc2)
    bc2p = jnp.zeros((1, D), f32).at[0, :2].set(bc2)

    cls_a, ts_a = _loss_call(
        acc2, h2, deg_t, ts_target.reshape(N, 1),
        b2.reshape(1, D), Wt1, bt1.reshape(1, D), wt2p, bt2p,
        Wc1, bc1.reshape(1, D), wc2p, bc2p,
    )
    return (cls_a[0, 0], ts_a[0, 0])


# speedup vs baseline: 25.0285x; 2.7867x over previous
import functools

import jax
import jax.numpy as jnp
from jax import lax
from jax.experimental import pallas as pl
from jax.experimental.pallas import tpu as pltpu
from jax.experimental.pallas import tpu_sc as plsc

N = 10000
E = 320000
D = 128
NTS = 5

NC = 2
NS = 16
NW = NC * NS
CH = 128
NCHUNK = 80
EPW = CH * NCHUNK
EPAD = EPW * NW
DUMP = N
NACC = 10240
RPT = NACC // NS
MCH = 2
MPAD = MCH * CH * NW

WIN = 16

RB = 2000


@functools.cache
def _sc_kernels():
    mesh = plsc.VectorSubcoreMesh(
        core_axis_name="c", subcore_axis_name="s", num_cores=NC, num_subcores=NS)

    @functools.partial(
        pl.kernel,
        out_type=jax.ShapeDtypeStruct((NC, NACC, D), jnp.float32),
        mesh=mesh,
        scratch_types=[
            pltpu.VMEM((NCHUNK, CH), jnp.int32),
            pltpu.VMEM((MCH, CH), jnp.int32),
            pltpu.VMEM((CH, D), jnp.float32),
            pltpu.VMEM((CH, D), jnp.float32),
            pltpu.VMEM_SHARED((NACC, D), jnp.float32),
        ],
    )
    def hist(dst_hbm, midx_hbm, ones_hbm, vm_hbm, zrow_hbm,
             tab_out,
             dst_v, midx_v, val_v, zb_v, tab_sp):
        c = lax.axis_index("c")
        s = lax.axis_index("s")
        w = c * NS + s
        pltpu.sync_copy(dst_hbm.at[w], dst_v)
        pltpu.sync_copy(midx_hbm.at[w], midx_v)
        pltpu.sync_copy(zrow_hbm, zb_v)
        base = s * RPT

        def zero_blk(k, carry):
            pltpu.sync_copy(zb_v, tab_sp.at[pl.ds(base + k * CH, CH)])
            return carry

        lax.fori_loop(0, RPT // CH, zero_blk, 0)
        pltpu.sync_copy(ones_hbm, val_v)
        plsc.subcore_barrier()

        def deg_blk(j, carry):
            pltpu.sync_copy(val_v, tab_sp.at[dst_v.at[j]], add=True)
            return carry

        lax.fori_loop(0, NCHUNK, deg_blk, 0)

        def mask_blk(j, carry):
            pltpu.sync_copy(vm_hbm.at[w, j], val_v)
            pltpu.sync_copy(val_v, tab_sp.at[midx_v.at[j]], add=True)
            return carry

        lax.fori_loop(0, MCH, mask_blk, 0)
        plsc.subcore_barrier()
        pltpu.sync_copy(tab_sp.at[pl.ds(base, RPT)],
                        tab_out.at[c, pl.ds(base, RPT)])

    @functools.partial(
        pl.kernel,
        out_type=jax.ShapeDtypeStruct((NC, NACC, D), jnp.float32),
        mesh=mesh,
        scratch_types=[
            pltpu.VMEM((WIN, CH), jnp.int32),
            pltpu.VMEM((WIN, CH), jnp.int32),
            pltpu.VMEM((CH, D), jnp.float32),
            pltpu.VMEM((CH, D), jnp.float32),
            pltpu.VMEM_SHARED((NACC, D), jnp.float32),
            pltpu.SemaphoreType.DMA,
            pltpu.SemaphoreType.DMA,
        ],
    )
    def segsum(tbl_hbm, src_hbm, dst_hbm, zrow_hbm, out_hbm,
               srcw_v, dstw_v, rows0, rows1, acc_sp, sem0, sem1):
        c = lax.axis_index("c")
        s = lax.axis_index("s")
        w = c * NS + s
        pltpu.sync_copy(zrow_hbm, rows0)
        base = s * RPT

        def zero_blk(k, carry):
            pltpu.sync_copy(rows0, acc_sp.at[pl.ds(base + k * CH, CH)])
            return carry

        lax.fori_loop(0, RPT // CH, zero_blk, 0)
        plsc.subcore_barrier()

        def window(win, carry):
            pltpu.sync_copy(src_hbm.at[w, pl.ds(win * WIN, WIN)], srcw_v)
            pltpu.sync_copy(dst_hbm.at[w, pl.ds(win * WIN, WIN)], dstw_v)
            pltpu.async_copy(tbl_hbm.at[srcw_v.at[0]], rows0, sem0)
            pltpu.async_copy(tbl_hbm.at[srcw_v.at[1]], rows1, sem1)

            def pair(jj, c2):
                a = 2 * jj
                b = a + 1
                pltpu.make_async_copy(
                    tbl_hbm.at[srcw_v.at[a]], rows0, sem0).wait()
                pltpu.sync_copy(rows0, acc_sp.at[dstw_v.at[a]], add=True)

                @pl.when(a + 2 < WIN)
                def _():
                    pltpu.async_copy(tbl_hbm.at[srcw_v.at[a + 2]], rows0, sem0)

                pltpu.make_async_copy(
                    tbl_hbm.at[srcw_v.at[b]], rows1, sem1).wait()
                pltpu.sync_copy(rows1, acc_sp.at[dstw_v.at[b]], add=True)

                @pl.when(b + 2 < WIN)
                def _():
                    pltpu.async_copy(tbl_hbm.at[srcw_v.at[b + 2]], rows1, sem1)

                return c2

            lax.fori_loop(0, WIN // 2, pair, 0)
            return carry

        lax.fori_loop(0, NCHUNK // WIN, window, 0)
        plsc.subcore_barrier()
        pltpu.sync_copy(acc_sp.at[pl.ds(base, RPT)],
                        out_hbm.at[c, pl.ds(base, RPT)])

    return hist, segsum


def _dinv_of(deg_ref):
    deg = deg_ref[0, :, 0:1] + deg_ref[1, :, 0:1] + 1.0
    return lax.rsqrt(deg)


def _h1_body(x_ref, w1_ref, deg_ref, out_ref):
    dinv = _dinv_of(deg_ref)
    h = jnp.dot(x_ref[:], w1_ref[:], preferred_element_type=jnp.float32)
    out_ref[:] = h * dinv


def _mid_body(acc_ref, h1_ref, deg_ref, b1_ref, w2_ref, out_ref):
    dinv = _dinv_of(deg_ref)
    zsum = acc_ref[0] + acc_ref[1] + h1_ref[:]
    z = jnp.maximum(dinv * zsum + b1_ref[:], 0.0)
    h2 = jnp.dot(z, w2_ref[:], preferred_element_type=jnp.float32)
    out_ref[:] = h2 * dinv


def _loss_body(acc_ref, h2_ref, deg_ref, tst_ref,
               b2_ref, wt1_ref, bt1_ref, wt2_ref, bt2_ref,
               wc1_ref, bc1_ref, wc2_ref, bc2_ref,
               cls_ref, ts_ref):
    i = pl.program_id(0)
    dinv = _dinv_of(deg_ref)
    zsum = acc_ref[0] + acc_ref[1] + h2_ref[:]
    z2 = dinv * zsum + b2_ref[:]

    col = lax.broadcasted_iota(jnp.int32, (1, D), 1)
    neg = jnp.float32(-1e30)

    t1 = jnp.maximum(
        jnp.dot(z2, wt1_ref[:], preferred_element_type=jnp.float32)
        + bt1_ref[:], 0.0)
    tl = jnp.dot(t1, wt2_ref[:], preferred_element_type=jnp.float32) + bt2_ref[:]
    tlm = jnp.where(col < NTS, tl, neg)
    m = jnp.max(tlm, axis=1, keepdims=True)
    se = jnp.sum(jnp.exp(tlm - m), axis=1, keepdims=True)
    lse = m + jnp.log(se)
    picked = jnp.sum(jnp.where(col == tst_ref[:], tl, 0.0), axis=1,
                     keepdims=True)
    ts_part = jnp.sum(lse - picked) * (1.0 / N)

    c1h = jnp.maximum(
        jnp.dot(z2, wc1_ref[:], preferred_element_type=jnp.float32)
        + bc1_ref[:], 0.0)
    cl = jnp.dot(c1h, wc2_ref[:], preferred_element_type=jnp.float32) + bc2_ref[:]
    clm = jnp.where(col < 2, cl, neg)
    m2 = jnp.max(clm, axis=1, keepdims=True)
    se2 = jnp.sum(jnp.exp(clm - m2), axis=1, keepdims=True)
    lse2 = m2 + jnp.log(se2)
    c0 = deg_ref[0, :, 1:2] + deg_ref[1, :, 1:2]
    c1 = deg_ref[0, :, 2:3] + deg_ref[1, :, 2:3]
    cls_part = jnp.sum((c0 + c1) * lse2 - c0 * cl[:, 0:1]
                       - c1 * cl[:, 1:2]) * (1.0 / 5000.0)

    @pl.when(i == 0)
    def _():
        cls_ref[:, :] = jnp.zeros((1, 1), jnp.float32)
        ts_ref[:, :] = jnp.zeros((1, 1), jnp.float32)

    cls_ref[:, :] += cls_part.reshape(1, 1)
    ts_ref[:, :] += ts_part.reshape(1, 1)


_h1_call = pl.pallas_call(
    _h1_body,
    grid=(N // RB,),
    in_specs=[
        pl.BlockSpec((RB, D), lambda i: (i, 0)),
        pl.BlockSpec((D, D), lambda i: (0, 0)),
        pl.BlockSpec((NC, RB, D), lambda i: (0, i, 0)),
    ],
    out_specs=pl.BlockSpec((RB, D), lambda i: (i, 0)),
    out_shape=jax.ShapeDtypeStruct((N, D), jnp.float32),
)

_mid_call = pl.pallas_call(
    _mid_body,
    grid=(N // RB,),
    in_specs=[
        pl.BlockSpec((NC, RB, D), lambda i: (0, i, 0)),
        pl.BlockSpec((RB, D), lambda i: (i, 0)),
        pl.BlockSpec((NC, RB, D), lambda i: (0, i, 0)),
        pl.BlockSpec((1, D), lambda i: (0, 0)),
        pl.BlockSpec((D, D), lambda i: (0, 0)),
    ],
    out_specs=pl.BlockSpec((RB, D), lambda i: (i, 0)),
    out_shape=jax.ShapeDtypeStruct((N, D), jnp.float32),
)

_loss_call = pl.pallas_call(
    _loss_body,
    grid=(N // RB,),
    in_specs=[
        pl.BlockSpec((NC, RB, D), lambda i: (0, i, 0)),
        pl.BlockSpec((RB, D), lambda i: (i, 0)),
        pl.BlockSpec((NC, RB, D), lambda i: (0, i, 0)),
        pl.BlockSpec((RB, 1), lambda i: (i, 0)),
        pl.BlockSpec((1, D), lambda i: (0, 0)),
        pl.BlockSpec((D, D), lambda i: (0, 0)),
        pl.BlockSpec((1, D), lambda i: (0, 0)),
        pl.BlockSpec((D, D), lambda i: (0, 0)),
        pl.BlockSpec((1, D), lambda i: (0, 0)),
        pl.BlockSpec((D, D), lambda i: (0, 0)),
        pl.BlockSpec((1, D), lambda i: (0, 0)),
        pl.BlockSpec((D, D), lambda i: (0, 0)),
        pl.BlockSpec((1, D), lambda i: (0, 0)),
    ],
    out_specs=[
        pl.BlockSpec((1, 1), lambda i: (0, 0)),
        pl.BlockSpec((1, 1), lambda i: (0, 0)),
    ],
    out_shape=[
        jax.ShapeDtypeStruct((1, 1), jnp.float32),
        jax.ShapeDtypeStruct((1, 1), jnp.float32),
    ],
)


def kernel(x, edge_index, ts_target, node_target, node_mask,
           W1, b1, W2, b2, Wt1, bt1, Wt2, bt2, Wc1, bc1, Wc2, bc2):
    f32 = jnp.float32
    i32 = jnp.int32
    hist, segsum = _sc_kernels()

    src = edge_index[0]
    dst = edge_index[1]
    epad = EPAD - E
    pad_ar = jnp.arange(epad, dtype=i32)
    src_pad = (pad_ar * 41) % N
    dst_pad = DUMP + (pad_ar % (NACC - N))
    srcp = jnp.concatenate([src, src_pad]).reshape(NW, NCHUNK, CH)
    dstp = jnp.concatenate([dst, dst_pad]).reshape(NW, NCHUNK, CH)

    mpad = MPAD - node_mask.shape[0]
    mpad_ar = jnp.arange(mpad, dtype=i32)
    midx = jnp.concatenate(
        [node_mask, DUMP + (mpad_ar % (NACC - N))]).reshape(NW, MCH, CH)
    lanes = jnp.arange(D, dtype=i32)
    lane0 = (lanes == 0).astype(f32)
    lane1 = (lanes == 1).astype(f32)
    lane2 = (lanes == 2).astype(f32)
    v0 = jnp.concatenate([(node_target == 0).astype(f32),
                          jnp.zeros((mpad,), f32)])
    v1 = jnp.concatenate([(node_target == 1).astype(f32),
                          jnp.zeros((mpad,), f32)])
    vmr = (v0[:, None] * lane1 + v1[:, None] * lane2).reshape(NW, MCH, CH, D)
    ones_rows = jnp.broadcast_to(lane0, (CH, D))
    zrow = jnp.zeros((CH, D), f32)

    deg_t = hist(dstp, midx, ones_rows, vmr, zrow)

    h1 = _h1_call(x, W1, deg_t)
    acc1 = segsum(h1, srcp, dstp, zrow)
    h2 = _mid_call(acc1, h1, deg_t, b1.reshape(1, D), W2)
    acc2 = segsum(h2, srcp, dstp, zrow)

    wt2p = jnp.zeros((D, D), f32).at[:, :NTS].set(Wt2)
    bt2p = jnp.zeros((1, D), f32).at[0, :NTS].set(bt2)
    wc2p = jnp.zeros((D, D), f32).at[:, :2].set(Wc2)
    bc2p = jnp.zeros((1, D), f32).at[0, :2].set(bc2)

    cls_a, ts_a = _loss_call(
        acc2, h2, deg_t, ts_target.reshape(N, 1),
        b2.reshape(1, D), Wt1, bt1.reshape(1, D), wt2p, bt2p,
        Wc1, bc1.reshape(1, D), wc2p, bc2p,
    )
    return (cls_a[0, 0], ts_a[0, 0])


# speedup vs baseline: 26.0890x; 1.0424x over previous
import functools

import jax
import jax.numpy as jnp
from jax import lax
from jax.experimental import pallas as pl
from jax.experimental.pallas import tpu as pltpu
from jax.experimental.pallas import tpu_sc as plsc

N = 10000
E = 320000
D = 128
NTS = 5

NC = 2
NS = 16
NW = NC * NS
CH = 128
NCHUNK = 80
EPW = CH * NCHUNK
EPAD = EPW * NW
DUMP = N
NACC = 10240
RPT = NACC // NS
MCH = 2
MPAD = MCH * CH * NW

WIN = 40

RB = 2000


@functools.cache
def _sc_kernels():
    mesh = plsc.VectorSubcoreMesh(
        core_axis_name="c", subcore_axis_name="s", num_cores=NC, num_subcores=NS)

    @functools.partial(
        pl.kernel,
        out_type=jax.ShapeDtypeStruct((NC, NACC, D), jnp.float32),
        mesh=mesh,
        scratch_types=[
            pltpu.VMEM((NCHUNK, CH), jnp.int32),
            pltpu.VMEM((MCH, CH), jnp.int32),
            pltpu.VMEM((CH, D), jnp.float32),
            pltpu.VMEM((CH, D), jnp.float32),
            pltpu.VMEM_SHARED((NACC, D), jnp.float32),
        ],
    )
    def hist(dst_hbm, midx_hbm, ones_hbm, vm_hbm, zrow_hbm,
             tab_out,
             dst_v, midx_v, val_v, zb_v, tab_sp):
        c = lax.axis_index("c")
        s = lax.axis_index("s")
        w = c * NS + s
        pltpu.sync_copy(dst_hbm.at[w], dst_v)
        pltpu.sync_copy(midx_hbm.at[w], midx_v)
        pltpu.sync_copy(zrow_hbm, zb_v)
        base = s * RPT

        def zero_blk(k, carry):
            pltpu.sync_copy(zb_v, tab_sp.at[pl.ds(base + k * CH, CH)])
            return carry

        lax.fori_loop(0, RPT // CH, zero_blk, 0)
        pltpu.sync_copy(ones_hbm, val_v)
        plsc.subcore_barrier()

        def deg_blk(j, carry):
            pltpu.sync_copy(val_v, tab_sp.at[dst_v.at[j]], add=True)
            return carry

        lax.fori_loop(0, NCHUNK, deg_blk, 0)

        def mask_blk(j, carry):
            pltpu.sync_copy(vm_hbm.at[w, j], val_v)
            pltpu.sync_copy(val_v, tab_sp.at[midx_v.at[j]], add=True)
            return carry

        lax.fori_loop(0, MCH, mask_blk, 0)
        plsc.subcore_barrier()
        pltpu.sync_copy(tab_sp.at[pl.ds(base, RPT)],
                        tab_out.at[c, pl.ds(base, RPT)])

    @functools.partial(
        pl.kernel,
        out_type=jax.ShapeDtypeStruct((NC, NACC, D), jnp.float32),
        mesh=mesh,
        scratch_types=[
            pltpu.VMEM((WIN, CH), jnp.int32),
            pltpu.VMEM((WIN, CH), jnp.int32),
            pltpu.VMEM((CH, D), jnp.float32),
            pltpu.VMEM((CH, D), jnp.float32),
            pltpu.VMEM_SHARED((NACC, D), jnp.float32),
            pltpu.SemaphoreType.DMA,
            pltpu.SemaphoreType.DMA,
        ],
    )
    def segsum(tbl_hbm, src_hbm, dst_hbm, zrow_hbm, out_hbm,
               srcw_v, dstw_v, rows0, rows1, acc_sp, sem0, sem1):
        c = lax.axis_index("c")
        s = lax.axis_index("s")
        w = c * NS + s
        pltpu.sync_copy(zrow_hbm, rows0)
        base = s * RPT

        def zero_blk(k, carry):
            pltpu.sync_copy(rows0, acc_sp.at[pl.ds(base + k * CH, CH)])
            return carry

        lax.fori_loop(0, RPT // CH, zero_blk, 0)
        plsc.subcore_barrier()

        def window(win, carry):
            pltpu.sync_copy(src_hbm.at[w, pl.ds(win * WIN, WIN)], srcw_v)
            pltpu.sync_copy(dst_hbm.at[w, pl.ds(win * WIN, WIN)], dstw_v)
            pltpu.async_copy(tbl_hbm.at[srcw_v.at[0]], rows0, sem0)
            pltpu.async_copy(tbl_hbm.at[srcw_v.at[1]], rows1, sem1)

            def pair(jj, c2):
                a = 2 * jj
                b = a + 1
                pltpu.make_async_copy(
                    tbl_hbm.at[srcw_v.at[a]], rows0, sem0).wait()
                pltpu.sync_copy(rows0, acc_sp.at[dstw_v.at[a]], add=True)

                @pl.when(a + 2 < WIN)
                def _():
                    pltpu.async_copy(tbl_hbm.at[srcw_v.at[a + 2]], rows0, sem0)

                pltpu.make_async_copy(
                    tbl_hbm.at[srcw_v.at[b]], rows1, sem1).wait()
                pltpu.sync_copy(rows1, acc_sp.at[dstw_v.at[b]], add=True)

                @pl.when(b + 2 < WIN)
                def _():
                    pltpu.async_copy(tbl_hbm.at[srcw_v.at[b + 2]], rows1, sem1)

                return c2

            lax.fori_loop(0, WIN // 2, pair, 0)
            return carry

        lax.fori_loop(0, NCHUNK // WIN, window, 0)
        plsc.subcore_barrier()
        pltpu.sync_copy(acc_sp.at[pl.ds(base, RPT)],
                        out_hbm.at[c, pl.ds(base, RPT)])

    return hist, segsum


def _dinv_of(deg_ref):
    deg = deg_ref[0, :, 0:1] + deg_ref[1, :, 0:1] + 1.0
    return lax.rsqrt(deg)


def _h1_body(x_ref, w1_ref, deg_ref, out_ref):
    dinv = _dinv_of(deg_ref)
    h = jnp.dot(x_ref[:], w1_ref[:], preferred_element_type=jnp.float32)
    out_ref[:] = h * dinv


def _mid_body(acc_ref, h1_ref, deg_ref, b1_ref, w2_ref, out_ref):
    dinv = _dinv_of(deg_ref)
    zsum = acc_ref[0] + acc_ref[1] + h1_ref[:]
    z = jnp.maximum(dinv * zsum + b1_ref[:], 0.0)
    h2 = jnp.dot(z, w2_ref[:], preferred_element_type=jnp.float32)
    out_ref[:] = h2 * dinv


def _loss_body(acc_ref, h2_ref, deg_ref, tst_ref,
               b2_ref, wt1_ref, bt1_ref, wt2_ref, bt2_ref,
               wc1_ref, bc1_ref, wc2_ref, bc2_ref,
               cls_ref, ts_ref):
    i = pl.program_id(0)
    dinv = _dinv_of(deg_ref)
    zsum = acc_ref[0] + acc_ref[1] + h2_ref[:]
    z2 = dinv * zsum + b2_ref[:]

    col = lax.broadcasted_iota(jnp.int32, (1, D), 1)
    neg = jnp.float32(-1e30)

    t1 = jnp.maximum(
        jnp.dot(z2, wt1_ref[:], preferred_element_type=jnp.float32)
        + bt1_ref[:], 0.0)
    tl = jnp.dot(t1, wt2_ref[:], preferred_element_type=jnp.float32) + bt2_ref[:]
    tlm = jnp.where(col < NTS, tl, neg)
    m = jnp.max(tlm, axis=1, keepdims=True)
    se = jnp.sum(jnp.exp(tlm - m), axis=1, keepdims=True)
    lse = m + jnp.log(se)
    picked = jnp.sum(jnp.where(col == tst_ref[:], tl, 0.0), axis=1,
                     keepdims=True)
    ts_part = jnp.sum(lse - picked) * (1.0 / N)

    c1h = jnp.maximum(
        jnp.dot(z2, wc1_ref[:], preferred_element_type=jnp.float32)
        + bc1_ref[:], 0.0)
    cl = jnp.dot(c1h, wc2_ref[:], preferred_element_type=jnp.float32) + bc2_ref[:]
    clm = jnp.where(col < 2, cl, neg)
    m2 = jnp.max(clm, axis=1, keepdims=True)
    se2 = jnp.sum(jnp.exp(clm - m2), axis=1, keepdims=True)
    lse2 = m2 + jnp.log(se2)
    c0 = deg_ref[0, :, 1:2] + deg_ref[1, :, 1:2]
    c1 = deg_ref[0, :, 2:3] + deg_ref[1, :, 2:3]
    cls_part = jnp.sum((c0 + c1) * lse2 - c0 * cl[:, 0:1]
                       - c1 * cl[:, 1:2]) * (1.0 / 5000.0)

    @pl.when(i == 0)
    def _():
        cls_ref[:, :] = jnp.zeros((1, 1), jnp.float32)
        ts_ref[:, :] = jnp.zeros((1, 1), jnp.float32)

    cls_ref[:, :] += cls_part.reshape(1, 1)
    ts_ref[:, :] += ts_part.reshape(1, 1)


_h1_call = pl.pallas_call(
    _h1_body,
    grid=(N // RB,),
    in_specs=[
        pl.BlockSpec((RB, D), lambda i: (i, 0)),
        pl.BlockSpec((D, D), lambda i: (0, 0)),
        pl.BlockSpec((NC, RB, D), lambda i: (0, i, 0)),
    ],
    out_specs=pl.BlockSpec((RB, D), lambda i: (i, 0)),
    out_shape=jax.ShapeDtypeStruct((N, D), jnp.float32),
)

_mid_call = pl.pallas_call(
    _mid_body,
    grid=(N // RB,),
    in_specs=[
        pl.BlockSpec((NC, RB, D), lambda i: (0, i, 0)),
        pl.BlockSpec((RB, D), lambda i: (i, 0)),
        pl.BlockSpec((NC, RB, D), lambda i: (0, i, 0)),
        pl.BlockSpec((1, D), lambda i: (0, 0)),
        pl.BlockSpec((D, D), lambda i: (0, 0)),
    ],
    out_specs=pl.BlockSpec((RB, D), lambda i: (i, 0)),
    out_shape=jax.ShapeDtypeStruct((N, D), jnp.float32),
)

_loss_call = pl.pallas_call(
    _loss_body,
    grid=(N // RB,),
    in_specs=[
        pl.BlockSpec((NC, RB, D), lambda i: (0, i, 0)),
        pl.BlockSpec((RB, D), lambda i: (i, 0)),
        pl.BlockSpec((NC, RB, D), lambda i: (0, i, 0)),
        pl.BlockSpec((RB, 1), lambda i: (i, 0)),
        pl.BlockSpec((1, D), lambda i: (0, 0)),
        pl.BlockSpec((D, D), lambda i: (0, 0)),
        pl.BlockSpec((1, D), lambda i: (0, 0)),
        pl.BlockSpec((D, D), lambda i: (0, 0)),
        pl.BlockSpec((1, D), lambda i: (0, 0)),
        pl.BlockSpec((D, D), lambda i: (0, 0)),
        pl.BlockSpec((1, D), lambda i: (0, 0)),
        pl.BlockSpec((D, D), lambda i: (0, 0)),
        pl.BlockSpec((1, D), lambda i: (0, 0)),
    ],
    out_specs=[
        pl.BlockSpec((1, 1), lambda i: (0, 0)),
        pl.BlockSpec((1, 1), lambda i: (0, 0)),
    ],
    out_shape=[
        jax.ShapeDtypeStruct((1, 1), jnp.float32),
        jax.ShapeDtypeStruct((1, 1), jnp.float32),
    ],
)


def kernel(x, edge_index, ts_target, node_target, node_mask,
           W1, b1, W2, b2, Wt1, bt1, Wt2, bt2, Wc1, bc1, Wc2, bc2):
    f32 = jnp.float32
    i32 = jnp.int32
    hist, segsum = _sc_kernels()

    src = edge_index[0]
    dst = edge_index[1]
    epad = EPAD - E
    pad_ar = jnp.arange(epad, dtype=i32)
    src_pad = (pad_ar * 41) % N
    dst_pad = DUMP + (pad_ar % (NACC - N))
    srcp = jnp.concatenate([src, src_pad]).reshape(NW, NCHUNK, CH)
    dstp = jnp.concatenate([dst, dst_pad]).reshape(NW, NCHUNK, CH)

    mpad = MPAD - node_mask.shape[0]
    mpad_ar = jnp.arange(mpad, dtype=i32)
    midx = jnp.concatenate(
        [node_mask, DUMP + (mpad_ar % (NACC - N))]).reshape(NW, MCH, CH)
    lanes = jnp.arange(D, dtype=i32)
    lane0 = (lanes == 0).astype(f32)
    lane1 = (lanes == 1).astype(f32)
    lane2 = (lanes == 2).astype(f32)
    v0 = jnp.concatenate([(node_target == 0).astype(f32),
                          jnp.zeros((mpad,), f32)])
    v1 = jnp.concatenate([(node_target == 1).astype(f32),
                          jnp.zeros((mpad,), f32)])
    vmr = (v0[:, None] * lane1 + v1[:, None] * lane2).reshape(NW, MCH, CH, D)
    ones_rows = jnp.broadcast_to(lane0, (CH, D))
    zrow = jnp.zeros((CH, D), f32)

    deg_t = hist(dstp, midx, ones_rows, vmr, zrow)

    h1 = _h1_call(x, W1, deg_t)
    acc1 = segsum(h1, srcp, dstp, zrow)
    h2 = _mid_call(acc1, h1, deg_t, b1.reshape(1, D), W2)
    acc2 = segsum(h2, srcp, dstp, zrow)

    wt2p = jnp.zeros((D, D), f32).at[:, :NTS].set(Wt2)
    bt2p = jnp.zeros((1, D), f32).at[0, :NTS].set(bt2)
    wc2p = jnp.zeros((D, D), f32).at[:, :2].set(Wc2)
    bc2p = jnp.zeros((1, D), f32).at[0, :2].set(bc2)

    cls_a, ts_a = _loss_call(
        acc2, h2, deg_t, ts_target.reshape(N, 1),
        b2.reshape(1, D), Wt1, bt1.reshape(1, D), wt2p, bt2p,
        Wc1, bc1.reshape(1, D), wc2p, bc2p,
    )
    return (cls_a[0, 0], ts_a[0, 0])


# speedup vs baseline: 26.2845x; 1.0075x over previous
import functools

import jax
import jax.numpy as jnp
from jax import lax
from jax.experimental import pallas as pl
from jax.experimental.pallas import tpu as pltpu
from jax.experimental.pallas import tpu_sc as plsc

N = 10000
E = 320000
D = 128
NTS = 5

NC = 2
NS = 16
NW = NC * NS
CH = 128
NCHUNK = 80
EPW = CH * NCHUNK
EPAD = EPW * NW
DUMP = N
NACC = 10240
RPT = NACC // NS
MCH = 2
MPAD = MCH * CH * NW

WIN = 40

RB = 2000


@functools.cache
def _sc_kernels():
    mesh = plsc.VectorSubcoreMesh(
        core_axis_name="c", subcore_axis_name="s", num_cores=NC, num_subcores=NS)

    @functools.partial(
        pl.kernel,
        out_type=jax.ShapeDtypeStruct((NC, NACC, D), jnp.float32),
        mesh=mesh,
        scratch_types=[
            pltpu.VMEM((NCHUNK, CH), jnp.int32),
            pltpu.VMEM((MCH, CH), jnp.int32),
            pltpu.VMEM((CH, D), jnp.float32),
            pltpu.VMEM((CH, D), jnp.float32),
            pltpu.VMEM_SHARED((NACC, D), jnp.float32),
        ],
    )
    def hist(dst_hbm, midx_hbm, ones_hbm, vm_hbm, zrow_hbm,
             tab_out,
             dst_v, midx_v, val_v, zb_v, tab_sp):
        c = lax.axis_index("c")
        s = lax.axis_index("s")
        w = c * NS + s
        pltpu.sync_copy(dst_hbm.at[w], dst_v)
        pltpu.sync_copy(midx_hbm.at[w], midx_v)
        pltpu.sync_copy(zrow_hbm, zb_v)
        base = s * RPT

        def zero_blk(k, carry):
            pltpu.sync_copy(zb_v, tab_sp.at[pl.ds(base + k * CH, CH)])
            return carry

        lax.fori_loop(0, RPT // CH, zero_blk, 0)
        pltpu.sync_copy(ones_hbm, val_v)
        plsc.subcore_barrier()

        def deg_blk(j, carry):
            pltpu.sync_copy(val_v, tab_sp.at[dst_v.at[j]], add=True)
            return carry

        lax.fori_loop(0, NCHUNK, deg_blk, 0)

        def mask_blk(j, carry):
            pltpu.sync_copy(vm_hbm.at[w, j], val_v)
            pltpu.sync_copy(val_v, tab_sp.at[midx_v.at[j]], add=True)
            return carry

        lax.fori_loop(0, MCH, mask_blk, 0)
        plsc.subcore_barrier()
        pltpu.sync_copy(tab_sp.at[pl.ds(base, RPT)],
                        tab_out.at[c, pl.ds(base, RPT)])

    @functools.partial(
        pl.kernel,
        out_type=jax.ShapeDtypeStruct((NC, NACC, D), jnp.float32),
        mesh=mesh,
        scratch_types=[
            pltpu.VMEM((WIN, CH), jnp.int32),
            pltpu.VMEM((WIN, CH), jnp.int32),
            pltpu.VMEM((CH, D), jnp.float32),
            pltpu.VMEM((CH, D), jnp.float32),
            pltpu.VMEM_SHARED((NACC, D), jnp.float32),
            pltpu.SemaphoreType.DMA,
            pltpu.SemaphoreType.DMA,
        ],
    )
    def segsum(tbl_hbm, src_hbm, dst_hbm, zrow_hbm, out_hbm,
               srcw_v, dstw_v, rows0, rows1, acc_sp, sem0, sem1):
        c = lax.axis_index("c")
        s = lax.axis_index("s")
        w = c * NS + s
        pltpu.sync_copy(zrow_hbm, rows0)
        base = s * RPT

        def zero_blk(k, carry):
            pltpu.sync_copy(rows0, acc_sp.at[pl.ds(base + k * CH, CH)])
            return carry

        lax.fori_loop(0, RPT // CH, zero_blk, 0)
        plsc.subcore_barrier()

        def window(win, carry):
            pltpu.sync_copy(src_hbm.at[w, pl.ds(win * WIN, WIN)], srcw_v)
            pltpu.sync_copy(dst_hbm.at[w, pl.ds(win * WIN, WIN)], dstw_v)
            pltpu.async_copy(tbl_hbm.at[srcw_v.at[0]], rows0, sem0)
            pltpu.async_copy(tbl_hbm.at[srcw_v.at[1]], rows1, sem1)

            def pair(jj, c2):
                a = 2 * jj
                b = a + 1
                pltpu.make_async_copy(
                    tbl_hbm.at[srcw_v.at[a]], rows0, sem0).wait()
                pltpu.sync_copy(rows0, acc_sp.at[dstw_v.at[a]], add=True)

                @pl.when(a + 2 < WIN)
                def _():
                    pltpu.async_copy(tbl_hbm.at[srcw_v.at[a + 2]], rows0, sem0)

                pltpu.make_async_copy(
                    tbl_hbm.at[srcw_v.at[b]], rows1, sem1).wait()
                pltpu.sync_copy(rows1, acc_sp.at[dstw_v.at[b]], add=True)

                @pl.when(b + 2 < WIN)
                def _():
                    pltpu.async_copy(tbl_hbm.at[srcw_v.at[b + 2]], rows1, sem1)

                return c2

            lax.fori_loop(0, WIN // 2, pair, 0)
            return carry

        lax.fori_loop(0, NCHUNK // WIN, window, 0)
        plsc.subcore_barrier()
        pltpu.sync_copy(acc_sp.at[pl.ds(base, RPT)],
                        out_hbm.at[c, pl.ds(base, RPT)])

    return hist, segsum


def _dinv_of(deg_ref):
    deg = deg_ref[0, :, 0:1] + deg_ref[1, :, 0:1] + 1.0
    return lax.rsqrt(deg)


def _h1raw_body(x_ref, w1_ref, out_ref):
    out_ref[:] = jnp.dot(x_ref[:], w1_ref[:],
                         preferred_element_type=jnp.float32)


def _scale_body(h_ref, deg_ref, out_ref, cnt_ref):
    dinv = _dinv_of(deg_ref)
    out_ref[:] = h_ref[:] * dinv
    lane = jnp.arange(8)
    c0 = deg_ref[0, :, 1:2] + deg_ref[1, :, 1:2]
    c1 = deg_ref[0, :, 2:3] + deg_ref[1, :, 2:3]
    cnt_ref[:] = dinv * (lane == 0) + c0 * (lane == 1) + c1 * (lane == 2)


def _mid_body(acc_ref, h1_ref, cnt_ref, b1_ref, w2_ref, out_ref):
    dinv = cnt_ref[:, 0:1]
    zsum = acc_ref[0] + acc_ref[1] + h1_ref[:]
    z = jnp.maximum(dinv * zsum + b1_ref[:], 0.0)
    h2 = jnp.dot(z, w2_ref[:], preferred_element_type=jnp.float32)
    out_ref[:] = h2 * dinv


def _loss_body(acc_ref, h2_ref, cnt_ref, tst_ref,
               b2_ref, wt1_ref, bt1_ref, wt2_ref, bt2_ref,
               wc1_ref, bc1_ref, wc2_ref, bc2_ref,
               cls_ref, ts_ref):
    i = pl.program_id(0)
    dinv = cnt_ref[:, 0:1]
    zsum = acc_ref[0] + acc_ref[1] + h2_ref[:]
    z2 = dinv * zsum + b2_ref[:]

    col = lax.broadcasted_iota(jnp.int32, (1, D), 1)
    neg = jnp.float32(-1e30)

    t1 = jnp.maximum(
        jnp.dot(z2, wt1_ref[:], preferred_element_type=jnp.float32)
        + bt1_ref[:], 0.0)
    tl = jnp.dot(t1, wt2_ref[:], preferred_element_type=jnp.float32) + bt2_ref[:]
    tlm = jnp.where(col < NTS, tl, neg)
    m = jnp.max(tlm, axis=1, keepdims=True)
    se = jnp.sum(jnp.exp(tlm - m), axis=1, keepdims=True)
    lse = m + jnp.log(se)
    picked = jnp.sum(jnp.where(col == tst_ref[:], tl, 0.0), axis=1,
                     keepdims=True)
    ts_part = jnp.sum(lse - picked) * (1.0 / N)

    c1h = jnp.maximum(
        jnp.dot(z2, wc1_ref[:], preferred_element_type=jnp.float32)
        + bc1_ref[:], 0.0)
    cl = jnp.dot(c1h, wc2_ref[:], preferred_element_type=jnp.float32) + bc2_ref[:]
    clm = jnp.where(col < 2, cl, neg)
    m2 = jnp.max(clm, axis=1, keepdims=True)
    se2 = jnp.sum(jnp.exp(clm - m2), axis=1, keepdims=True)
    lse2 = m2 + jnp.log(se2)
    c0 = cnt_ref[:, 1:2]
    c1 = cnt_ref[:, 2:3]
    cls_part = jnp.sum((c0 + c1) * lse2 - c0 * cl[:, 0:1]
                       - c1 * cl[:, 1:2]) * (1.0 / 5000.0)

    @pl.when(i == 0)
    def _():
        cls_ref[:, :] = jnp.zeros((1, 1), jnp.float32)
        ts_ref[:, :] = jnp.zeros((1, 1), jnp.float32)

    cls_ref[:, :] += cls_part.reshape(1, 1)
    ts_ref[:, :] += ts_part.reshape(1, 1)


_h1raw_call = pl.pallas_call(
    _h1raw_body,
    grid=(N // RB,),
    in_specs=[
        pl.BlockSpec((RB, D), lambda i: (i, 0)),
        pl.BlockSpec((D, D), lambda i: (0, 0)),
    ],
    out_specs=pl.BlockSpec((RB, D), lambda i: (i, 0)),
    out_shape=jax.ShapeDtypeStruct((N, D), jnp.float32),
)

_scale_call = pl.pallas_call(
    _scale_body,
    grid=(N // RB,),
    in_specs=[
        pl.BlockSpec((RB, D), lambda i: (i, 0)),
        pl.BlockSpec((NC, RB, D), lambda i: (0, i, 0)),
    ],
    out_specs=[
        pl.BlockSpec((RB, D), lambda i: (i, 0)),
        pl.BlockSpec((RB, 8), lambda i: (i, 0)),
    ],
    out_shape=[
        jax.ShapeDtypeStruct((N, D), jnp.float32),
        jax.ShapeDtypeStruct((N, 8), jnp.float32),
    ],
)

_mid_call = pl.pallas_call(
    _mid_body,
    grid=(N // RB,),
    in_specs=[
        pl.BlockSpec((NC, RB, D), lambda i: (0, i, 0)),
        pl.BlockSpec((RB, D), lambda i: (i, 0)),
        pl.BlockSpec((RB, 8), lambda i: (i, 0)),
        pl.BlockSpec((1, D), lambda i: (0, 0)),
        pl.BlockSpec((D, D), lambda i: (0, 0)),
    ],
    out_specs=pl.BlockSpec((RB, D), lambda i: (i, 0)),
    out_shape=jax.ShapeDtypeStruct((N, D), jnp.float32),
)

_loss_call = pl.pallas_call(
    _loss_body,
    grid=(N // RB,),
    in_specs=[
        pl.BlockSpec((NC, RB, D), lambda i: (0, i, 0)),
        pl.BlockSpec((RB, D), lambda i: (i, 0)),
        pl.BlockSpec((RB, 8), lambda i: (i, 0)),
        pl.BlockSpec((RB, 1), lambda i: (i, 0)),
        pl.BlockSpec((1, D), lambda i: (0, 0)),
        pl.BlockSpec((D, D), lambda i: (0, 0)),
        pl.BlockSpec((1, D), lambda i: (0, 0)),
        pl.BlockSpec((D, D), lambda i: (0, 0)),
        pl.BlockSpec((1, D), lambda i: (0, 0)),
        pl.BlockSpec((D, D), lambda i: (0, 0)),
        pl.BlockSpec((1, D), lambda i: (0, 0)),
        pl.BlockSpec((D, D), lambda i: (0, 0)),
        pl.BlockSpec((1, D), lambda i: (0, 0)),
    ],
    out_specs=[
        pl.BlockSpec((1, 1), lambda i: (0, 0)),
        pl.BlockSpec((1, 1), lambda i: (0, 0)),
    ],
    out_shape=[
        jax.ShapeDtypeStruct((1, 1), jnp.float32),
        jax.ShapeDtypeStruct((1, 1), jnp.float32),
    ],
)


def kernel(x, edge_index, ts_target, node_target, node_mask,
           W1, b1, W2, b2, Wt1, bt1, Wt2, bt2, Wc1, bc1, Wc2, bc2):
    f32 = jnp.float32
    i32 = jnp.int32
    hist, segsum = _sc_kernels()

    src = edge_index[0]
    dst = edge_index[1]
    epad = EPAD - E
    pad_ar = jnp.arange(epad, dtype=i32)
    src_pad = (pad_ar * 41) % N
    dst_pad = DUMP + (pad_ar % (NACC - N))
    srcp = jnp.concatenate([src, src_pad]).reshape(NW, NCHUNK, CH)
    dstp = jnp.concatenate([dst, dst_pad]).reshape(NW, NCHUNK, CH)

    mpad = MPAD - node_mask.shape[0]
    mpad_ar = jnp.arange(mpad, dtype=i32)
    midx = jnp.concatenate(
        [node_mask, DUMP + (mpad_ar % (NACC - N))]).reshape(NW, MCH, CH)
    lanes = jnp.arange(D, dtype=i32)
    lane0 = (lanes == 0).astype(f32)
    lane1 = (lanes == 1).astype(f32)
    lane2 = (lanes == 2).astype(f32)
    v0 = jnp.concatenate([(node_target == 0).astype(f32),
                          jnp.zeros((mpad,), f32)])
    v1 = jnp.concatenate([(node_target == 1).astype(f32),
                          jnp.zeros((mpad,), f32)])
    vmr = (v0[:, None] * lane1 + v1[:, None] * lane2).reshape(NW, MCH, CH, D)
    ones_rows = jnp.broadcast_to(lane0, (CH, D))
    zrow = jnp.zeros((CH, D), f32)

    h1raw = _h1raw_call(x, W1)
    deg_t = hist(dstp, midx, ones_rows, vmr, zrow)
    h1, cnt = _scale_call(h1raw, deg_t)
    acc1 = segsum(h1, srcp, dstp, zrow)
    h2 = _mid_call(acc1, h1, cnt, b1.reshape(1, D), W2)
    acc2 = segsum(h2, srcp, dstp, zrow)

    wt2p = jnp.zeros((D, D), f32).at[:, :NTS].set(Wt2)
    bt2p = jnp.zeros((1, D), f32).at[0, :NTS].set(bt2)
    wc2p = jnp.zeros((D, D), f32).at[:, :2].set(Wc2)
    bc2p = jnp.zeros((1, D), f32).at[0, :2].set(bc2)

    cls_a, ts_a = _loss_call(
        acc2, h2, cnt, ts_target.reshape(N, 1),
        b2.reshape(1, D), Wt1, bt1.reshape(1, D), wt2p, bt2p,
        Wc1, bc1.reshape(1, D), wc2p, bc2p,
    )
    return (cls_a[0, 0], ts_a[0, 0])
